# Initial kernel scaffold; baseline (speedup 1.0000x reference)
import jax
import jax.numpy as jnp
from jax.experimental import pallas as pl


def kernel(x, conv_w, conv_b, fc_w, fc_b):
    raise NotImplementedError("write your pallas kernel here")


# speedup vs baseline: 1.2204x; 1.2204x over previous
import functools

import jax
import jax.numpy as jnp
from jax import lax
from jax.experimental import pallas as pl
from jax.experimental.pallas import tpu as pltpu

C_OUT = 10
OH = OW = 14
HW = OH * OW
N_TAPS = 9
N_CLASSES = 10
LANES = 128
CH_STRIDE = 256
K_CAT = C_OUT * CH_STRIDE


def _cnn_kernel(p_ref, cw_ref, cb_ref, fcw_ref, fcb_ref, o_ref, act_ref, *,
                sub_n):
    tb = o_ref.shape[0]
    n_sub = tb // sub_n

    pad_zeros = jnp.zeros((tb, CH_STRIDE - HW), jnp.float32)
    for c in range(C_OUT):
        act_ref[:, c * CH_STRIDE + HW:(c + 1) * CH_STRIDE] = pad_zeros

    @pl.loop(0, n_sub)
    def _(s):
        off = pl.multiple_of(s * sub_n, 8)
        taps = [p_ref[t, pl.ds(off, sub_n), :] for t in range(N_TAPS)]
        for c in range(C_OUT):
            acc = taps[0] * cw_ref[c, 0]
            for t in range(1, N_TAPS):
                acc = acc + taps[t] * cw_ref[c, t]
            act_ref[pl.ds(off, sub_n), c * CH_STRIDE:c * CH_STRIDE + HW] = (
                jnp.maximum(acc + cb_ref[c], 0.0))

    logits = jnp.dot(act_ref[...], fcw_ref[...],
                     preferred_element_type=jnp.float32) + fcb_ref[...]

    lane = lax.broadcasted_iota(jnp.int32, (tb, LANES), 1)
    valid = lane < N_CLASSES
    masked = jnp.where(valid, logits, -jnp.inf)
    m = jnp.max(masked, axis=-1, keepdims=True)
    e = jnp.exp(masked - m)
    lse = jnp.log(jnp.sum(e, axis=-1, keepdims=True))
    o_ref[...] = jnp.where(valid, logits - m - lse, 0.0)


def _round_up(a, m):
    return ((a + m - 1) // m) * m


def _build_taps(x):
    n = x.shape[0]
    x2 = x.astype(jnp.float32).reshape(n, 28, 28)
    x_pad = jnp.pad(x2, ((0, 0), (1, 1), (1, 1)))
    taps = [x_pad[:, di:di + 28:2, dj:dj + 28:2]
            for di in range(3) for dj in range(3)]
    return jnp.stack(taps, axis=0).reshape(N_TAPS, n, HW)


@jax.jit
def _forward(x, conv_w, conv_b, fc_w, fc_b):
    n = x.shape[0]
    patches = _build_taps(x)

    tb = 512
    sub = 64
    n_pad = _round_up(max(n, 2 * tb), tb)
    if n_pad != n:
        patches = jnp.pad(patches, ((0, 0), (0, n_pad - n), (0, 0)))

    cw = conv_w.reshape(C_OUT, N_TAPS).astype(jnp.float32)
    cb = conv_b.astype(jnp.float32)
    fcw_r = jnp.transpose(
        fc_w.astype(jnp.float32).reshape(N_CLASSES, C_OUT, HW), (1, 2, 0))
    fcw_p = jnp.pad(fcw_r, ((0, 0), (0, CH_STRIDE - HW), (0, LANES - N_CLASSES)))
    fcw_cat = fcw_p.reshape(K_CAT, LANES)
    fcb_p = jnp.pad(fc_b.astype(jnp.float32),
                    (0, LANES - N_CLASSES)).reshape(1, LANES)

    grid = (n_pad // tb,)
    flops = n_pad * (2 * HW * N_TAPS * C_OUT + 2 * K_CAT * LANES)
    bytes_accessed = int(4 * (patches.size + cw.size + cb.size
                              + fcw_cat.size + fcb_p.size + n_pad * LANES))

    out = pl.pallas_call(
        functools.partial(_cnn_kernel, sub_n=sub),
        out_shape=jax.ShapeDtypeStruct((n_pad, LANES), jnp.float32),
        grid=grid,
        in_specs=[
            pl.BlockSpec((N_TAPS, tb, HW), lambda i: (0, i, 0)),
            pl.BlockSpec(memory_space=pltpu.MemorySpace.SMEM),
            pl.BlockSpec(memory_space=pltpu.MemorySpace.SMEM),
            pl.BlockSpec((K_CAT, LANES), lambda i: (0, 0)),
            pl.BlockSpec((1, LANES), lambda i: (0, 0)),
        ],
        out_specs=pl.BlockSpec((tb, LANES), lambda i: (i, 0)),
        scratch_shapes=[pltpu.VMEM((tb, K_CAT), jnp.float32)],
        compiler_params=pltpu.CompilerParams(
            dimension_semantics=("parallel",),
            allow_input_fusion=[True, False, False, False, False]),
        cost_estimate=pl.CostEstimate(
            flops=flops,
            transcendentals=n_pad * (LANES + 1),
            bytes_accessed=bytes_accessed),
    )(patches, cw, cb, fcw_cat, fcb_p)

    return out[:n, :N_CLASSES]


def kernel(x, conv_w, conv_b, fc_w, fc_b):
    return _forward(x, conv_w, conv_b, fc_w, fc_b)


# speedup vs baseline: 3.9278x; 3.2184x over previous
import functools

import numpy as np

import jax
import jax.numpy as jnp
from jax import lax
from jax.experimental import pallas as pl
from jax.experimental.pallas import tpu as pltpu

C_OUT = 10
OH = OW = 14
HW = OH * OW
IN_HW = 28 * 28
N_TAPS = 9
N_CLASSES = 10
LANES = 128
CH_STRIDE = 256
K_CAT = C_OUT * CH_STRIDE


def _selection_tensor():
    s = np.zeros((N_TAPS, IN_HW, HW), np.float32)
    for di in range(3):
        for dj in range(3):
            t = di * 3 + dj
            for i in range(OH):
                r = 2 * i + di - 1
                if not 0 <= r < 28:
                    continue
                for j in range(OW):
                    cc = 2 * j + dj - 1
                    if not 0 <= cc < 28:
                        continue
                    s[t, r * 28 + cc, i * OW + j] = 1.0
    return s


_SEL = _selection_tensor()


def _cnn_kernel(x_ref, cw_ref, cb_ref, fcw_ref, fcb_ref, o_ref, act_ref):
    tb = o_ref.shape[0]

    conv = jnp.dot(x_ref[...], cw_ref[...], preferred_element_type=jnp.float32)
    act_ref[...] = jnp.maximum(conv + cb_ref[...], 0.0)

    logits = jnp.dot(act_ref[...], fcw_ref[...],
                     preferred_element_type=jnp.float32) + fcb_ref[...]

    lane = lax.broadcasted_iota(jnp.int32, (tb, LANES), 1)
    valid = lane < N_CLASSES
    masked = jnp.where(valid, logits, -jnp.inf)
    m = jnp.max(masked, axis=-1, keepdims=True)
    e = jnp.exp(masked - m)
    lse = jnp.log(jnp.sum(e, axis=-1, keepdims=True))
    o_ref[...] = jnp.where(valid, logits - m - lse, 0.0)


def _round_up(a, m):
    return ((a + m - 1) // m) * m


@jax.jit
def _forward(x, conv_w, conv_b, fc_w, fc_b):
    n = x.shape[0]
    xf = x.astype(jnp.float32).reshape(n, IN_HW)

    tb = 512
    n_pad = _round_up(max(n, 2 * tb), tb)
    if n_pad != n:
        xf = jnp.pad(xf, ((0, n_pad - n), (0, 0)))

    cw = conv_w.astype(jnp.float32).reshape(C_OUT, N_TAPS)
    cw_cat = jnp.einsum('tph,ct->pch', jnp.asarray(_SEL), cw)
    cw_cat = jnp.pad(cw_cat, ((0, 0), (0, 0), (0, CH_STRIDE - HW)))
    cw_cat = cw_cat.reshape(IN_HW, K_CAT)

    cb_cat = jnp.pad(
        jnp.broadcast_to(conv_b.astype(jnp.float32)[:, None], (C_OUT, HW)),
        ((0, 0), (0, CH_STRIDE - HW))).reshape(1, K_CAT)

    fcw_r = jnp.transpose(
        fc_w.astype(jnp.float32).reshape(N_CLASSES, C_OUT, HW), (1, 2, 0))
    fcw_p = jnp.pad(fcw_r, ((0, 0), (0, CH_STRIDE - HW), (0, LANES - N_CLASSES)))
    fcw_cat = fcw_p.reshape(K_CAT, LANES)
    fcb_p = jnp.pad(fc_b.astype(jnp.float32),
                    (0, LANES - N_CLASSES)).reshape(1, LANES)

    grid = (n_pad // tb,)
    flops = n_pad * (2 * IN_HW * K_CAT + 2 * K_CAT * LANES)
    bytes_accessed = int(4 * (xf.size + cw_cat.size + cb_cat.size
                              + fcw_cat.size + fcb_p.size + n_pad * LANES))

    out = pl.pallas_call(
        _cnn_kernel,
        out_shape=jax.ShapeDtypeStruct((n_pad, LANES), jnp.float32),
        grid=grid,
        in_specs=[
            pl.BlockSpec((tb, IN_HW), lambda i: (i, 0)),
            pl.BlockSpec((IN_HW, K_CAT), lambda i: (0, 0)),
            pl.BlockSpec((1, K_CAT), lambda i: (0, 0)),
            pl.BlockSpec((K_CAT, LANES), lambda i: (0, 0)),
            pl.BlockSpec((1, LANES), lambda i: (0, 0)),
        ],
        out_specs=pl.BlockSpec((tb, LANES), lambda i: (i, 0)),
        scratch_shapes=[pltpu.VMEM((tb, K_CAT), jnp.float32)],
        compiler_params=pltpu.CompilerParams(
            dimension_semantics=("parallel",)),
        cost_estimate=pl.CostEstimate(
            flops=flops,
            transcendentals=n_pad * (LANES + 1),
            bytes_accessed=bytes_accessed),
    )(xf, cw_cat, cb_cat, fcw_cat, fcb_p)

    return out[:n, :N_CLASSES]


def kernel(x, conv_w, conv_b, fc_w, fc_b):
    return _forward(x, conv_w, conv_b, fc_w, fc_b)


# speedup vs baseline: 4.3725x; 1.1132x over previous
import functools

import jax
import jax.numpy as jnp
from jax import lax
from jax.experimental import pallas as pl
from jax.experimental.pallas import tpu as pltpu

C_OUT = 10
OH = OW = 14
HW = OH * OW
IN_HW = 28 * 28
N_TAPS = 9
N_CLASSES = 10
LANES = 128
CH_STRIDE = 256
K_CAT = C_OUT * CH_STRIDE


def _cw_builder_kernel(cw_ref, out_ref):
    c = pl.program_id(0)
    row = lax.broadcasted_iota(jnp.int32, (IN_HW, CH_STRIDE), 0)
    col = lax.broadcasted_iota(jnp.int32, (IN_HW, CH_STRIDE), 1)
    pr = row // 28
    pc = row - 28 * pr
    oi = col // OW
    oj = col - OW * oi
    valid = col < HW
    acc = jnp.zeros((IN_HW, CH_STRIDE), jnp.float32)
    for di in range(3):
        for dj in range(3):
            m = ((pr == 2 * oi + di - 1) & (pc == 2 * oj + dj - 1)) & valid
            acc = jnp.where(m, cw_ref[c, di * 3 + dj], acc)
    out_ref[...] = acc


def _cnn_kernel(x_ref, cw_ref, cb_ref, fcw_ref, fcb_ref, o_ref, act_ref):
    tb = o_ref.shape[0]

    conv = jnp.dot(x_ref[...], cw_ref[...], preferred_element_type=jnp.float32)
    act_ref[...] = jnp.maximum(conv + cb_ref[...], 0.0)

    logits = jnp.dot(act_ref[...], fcw_ref[...],
                     preferred_element_type=jnp.float32) + fcb_ref[...]

    lane = lax.broadcasted_iota(jnp.int32, (tb, LANES), 1)
    valid = lane < N_CLASSES
    masked = jnp.where(valid, logits, -jnp.inf)
    m = jnp.max(masked, axis=-1, keepdims=True)
    e = jnp.exp(masked - m)
    lse = jnp.log(jnp.sum(e, axis=-1, keepdims=True))
    o_ref[...] = jnp.where(valid, logits - m - lse, 0.0)


def _round_up(a, m):
    return ((a + m - 1) // m) * m


@jax.jit
def _forward(x, conv_w, conv_b, fc_w, fc_b):
    n = x.shape[0]
    xf = x.astype(jnp.float32).reshape(n, IN_HW)

    tb = 512
    n_pad = _round_up(max(n, 2 * tb), tb)
    if n_pad != n:
        xf = jnp.pad(xf, ((0, n_pad - n), (0, 0)))

    cw = conv_w.astype(jnp.float32).reshape(C_OUT, N_TAPS)

    cw_cat = pl.pallas_call(
        _cw_builder_kernel,
        out_shape=jax.ShapeDtypeStruct((IN_HW, K_CAT), jnp.float32),
        grid=(C_OUT,),
        in_specs=[pl.BlockSpec(memory_space=pltpu.MemorySpace.SMEM)],
        out_specs=pl.BlockSpec((IN_HW, CH_STRIDE), lambda c: (0, c)),
        compiler_params=pltpu.CompilerParams(
            dimension_semantics=("parallel",)),
    )(cw)

    cb_cat = jnp.pad(
        jnp.broadcast_to(conv_b.astype(jnp.float32)[:, None], (C_OUT, HW)),
        ((0, 0), (0, CH_STRIDE - HW))).reshape(1, K_CAT)

    fcw_r = jnp.transpose(
        fc_w.astype(jnp.float32).reshape(N_CLASSES, C_OUT, HW), (1, 2, 0))
    fcw_p = jnp.pad(fcw_r, ((0, 0), (0, CH_STRIDE - HW), (0, LANES - N_CLASSES)))
    fcw_cat = fcw_p.reshape(K_CAT, LANES)
    fcb_p = jnp.pad(fc_b.astype(jnp.float32),
                    (0, LANES - N_CLASSES)).reshape(1, LANES)

    grid = (n_pad // tb,)
    flops = n_pad * (2 * IN_HW * K_CAT + 2 * K_CAT * LANES)
    bytes_accessed = int(4 * (xf.size + cw_cat.size + cb_cat.size
                              + fcw_cat.size + fcb_p.size + n_pad * LANES))

    out = pl.pallas_call(
        _cnn_kernel,
        out_shape=jax.ShapeDtypeStruct((n_pad, LANES), jnp.float32),
        grid=grid,
        in_specs=[
            pl.BlockSpec((tb, IN_HW), lambda i: (i, 0)),
            pl.BlockSpec((IN_HW, K_CAT), lambda i: (0, 0)),
            pl.BlockSpec((1, K_CAT), lambda i: (0, 0)),
            pl.BlockSpec((K_CAT, LANES), lambda i: (0, 0)),
            pl.BlockSpec((1, LANES), lambda i: (0, 0)),
        ],
        out_specs=pl.BlockSpec((tb, LANES), lambda i: (i, 0)),
        scratch_shapes=[pltpu.VMEM((tb, K_CAT), jnp.float32)],
        compiler_params=pltpu.CompilerParams(
            dimension_semantics=("parallel",)),
        cost_estimate=pl.CostEstimate(
            flops=flops,
            transcendentals=n_pad * (LANES + 1),
            bytes_accessed=bytes_accessed),
    )(xf, cw_cat, cb_cat, fcw_cat, fcb_p)

    return out[:n, :N_CLASSES]


def kernel(x, conv_w, conv_b, fc_w, fc_b):
    return _forward(x, conv_w, conv_b, fc_w, fc_b)


# speedup vs baseline: 4.4572x; 1.0194x over previous
import functools

import jax
import jax.numpy as jnp
from jax import lax
from jax.experimental import pallas as pl
from jax.experimental.pallas import tpu as pltpu

C_OUT = 10
OH = OW = 14
HW = OH * OW
IN_HW = 28 * 28
N_TAPS = 9
N_CLASSES = 10
LANES = 128
CH_STRIDE = 256
K_CAT = C_OUT * CH_STRIDE


def _cw_builder_kernel(cw_ref, out_ref):
    c = pl.program_id(0)
    row = lax.broadcasted_iota(jnp.int32, (IN_HW, CH_STRIDE), 0)
    col = lax.broadcasted_iota(jnp.int32, (IN_HW, CH_STRIDE), 1)
    pr = row // 28
    pc = row - 28 * pr
    oi = col // OW
    oj = col - OW * oi
    valid = col < HW
    acc = jnp.zeros((IN_HW, CH_STRIDE), jnp.float32)
    for di in range(3):
        for dj in range(3):
            m = ((pr == 2 * oi + di - 1) & (pc == 2 * oj + dj - 1)) & valid
            acc = jnp.where(m, cw_ref[c, di * 3 + dj], acc)
    out_ref[...] = acc


def _cnn_kernel(x_ref, cw_ref, cb_ref, fcw_ref, fcb_ref, o_ref, act_ref):
    tb = o_ref.shape[0]

    conv = jnp.dot(x_ref[...], cw_ref[...], preferred_element_type=jnp.float32)
    act_ref[...] = jnp.maximum(conv + cb_ref[...], 0.0)

    logits = jnp.dot(act_ref[...], fcw_ref[...],
                     preferred_element_type=jnp.float32) + fcb_ref[...]

    lane = lax.broadcasted_iota(jnp.int32, (tb, LANES), 1)
    valid = lane < N_CLASSES
    masked = jnp.where(valid, logits, -jnp.inf)
    m = jnp.max(masked, axis=-1, keepdims=True)
    e = jnp.exp(masked - m)
    lse = jnp.log(jnp.sum(e, axis=-1, keepdims=True))
    o_ref[...] = jnp.where(valid, logits - m - lse, 0.0)


def _round_up(a, m):
    return ((a + m - 1) // m) * m


@jax.jit
def _forward(x, conv_w, conv_b, fc_w, fc_b):
    n = x.shape[0]
    xf = x.astype(jnp.float32).reshape(n, IN_HW)

    tb = 2048
    n_pad = _round_up(max(n, 2 * tb), tb)
    if n_pad != n:
        xf = jnp.pad(xf, ((0, n_pad - n), (0, 0)))

    cw = conv_w.astype(jnp.float32).reshape(C_OUT, N_TAPS)

    cw_cat = pl.pallas_call(
        _cw_builder_kernel,
        out_shape=jax.ShapeDtypeStruct((IN_HW, K_CAT), jnp.float32),
        grid=(C_OUT,),
        in_specs=[pl.BlockSpec(memory_space=pltpu.MemorySpace.SMEM)],
        out_specs=pl.BlockSpec((IN_HW, CH_STRIDE), lambda c: (0, c)),
        compiler_params=pltpu.CompilerParams(
            dimension_semantics=("parallel",)),
    )(cw)

    cb_cat = jnp.pad(
        jnp.broadcast_to(conv_b.astype(jnp.float32)[:, None], (C_OUT, HW)),
        ((0, 0), (0, CH_STRIDE - HW))).reshape(1, K_CAT)

    fcw_r = jnp.transpose(
        fc_w.astype(jnp.float32).reshape(N_CLASSES, C_OUT, HW), (1, 2, 0))
    fcw_p = jnp.pad(fcw_r, ((0, 0), (0, CH_STRIDE - HW), (0, LANES - N_CLASSES)))
    fcw_cat = fcw_p.reshape(K_CAT, LANES)
    fcb_p = jnp.pad(fc_b.astype(jnp.float32),
                    (0, LANES - N_CLASSES)).reshape(1, LANES)

    grid = (n_pad // tb,)
    flops = n_pad * (2 * IN_HW * K_CAT + 2 * K_CAT * LANES)
    bytes_accessed = int(4 * (xf.size + cw_cat.size + cb_cat.size
                              + fcw_cat.size + fcb_p.size + n_pad * LANES))

    out = pl.pallas_call(
        _cnn_kernel,
        out_shape=jax.ShapeDtypeStruct((n_pad, LANES), jnp.float32),
        grid=grid,
        in_specs=[
            pl.BlockSpec((tb, IN_HW), lambda i: (i, 0)),
            pl.BlockSpec((IN_HW, K_CAT), lambda i: (0, 0)),
            pl.BlockSpec((1, K_CAT), lambda i: (0, 0)),
            pl.BlockSpec((K_CAT, LANES), lambda i: (0, 0)),
            pl.BlockSpec((1, LANES), lambda i: (0, 0)),
        ],
        out_specs=pl.BlockSpec((tb, LANES), lambda i: (i, 0)),
        scratch_shapes=[pltpu.VMEM((tb, K_CAT), jnp.float32)],
        compiler_params=pltpu.CompilerParams(
            dimension_semantics=("parallel",)),
        cost_estimate=pl.CostEstimate(
            flops=flops,
            transcendentals=n_pad * (LANES + 1),
            bytes_accessed=bytes_accessed),
    )(xf, cw_cat, cb_cat, fcw_cat, fcb_p)

    return out[:n, :N_CLASSES]


def kernel(x, conv_w, conv_b, fc_w, fc_b):
    return _forward(x, conv_w, conv_b, fc_w, fc_b)


# speedup vs baseline: 5.2269x; 1.1727x over previous
import functools

import jax
import jax.numpy as jnp
from jax import lax
from jax.experimental import pallas as pl
from jax.experimental.pallas import tpu as pltpu

C_OUT = 10
OH = OW = 14
HW = OH * OW
IN_HW = 28 * 28
N_TAPS = 9
N_CLASSES = 10
ROW_STRIDE = 16
K_CAT = C_OUT * OH * ROW_STRIDE
M_PAD = 16


def _cnn_kernel(x_ref, cw_ref, cb_ref, fcw_ref, fcb_ref, o_ref,
                xp_ref, act_ref):
    nb = o_ref.shape[1]
    n_q = nb // 128

    xp_ref[...] = jnp.zeros(xp_ref.shape, jnp.float32)
    for q in range(n_q):
        for r in range(28):
            xp_ref[q, r + 1, 1:29, :] = (
                x_ref[28 * r:28 * r + 28, 128 * q:128 * (q + 1)])

    zero2 = jnp.zeros((ROW_STRIDE - OW, 128), jnp.float32)
    for q in range(n_q):
        for i in range(OH):
            taps = [xp_ref[q, 2 * i + di, dj:dj + 27:2, :]
                    for di in range(3) for dj in range(3)]
            for c in range(C_OUT):
                acc = taps[0] * cw_ref[c, 0]
                for t in range(1, N_TAPS):
                    acc = acc + taps[t] * cw_ref[c, t]
                base = c * (OH * ROW_STRIDE) + i * ROW_STRIDE
                act_ref[pl.ds(base, OW), 128 * q:128 * (q + 1)] = (
                    jnp.maximum(acc + cb_ref[c], 0.0))
                act_ref[pl.ds(base + OW, ROW_STRIDE - OW),
                        128 * q:128 * (q + 1)] = zero2

    logits = jnp.dot(fcw_ref[...], act_ref[...],
                     preferred_element_type=jnp.float32) + fcb_ref[:, 0:1]

    row = lax.broadcasted_iota(jnp.int32, (M_PAD, nb), 0)
    valid = row < N_CLASSES
    masked = jnp.where(valid, logits, -jnp.inf)
    m = jnp.max(masked, axis=0, keepdims=True)
    e = jnp.exp(masked - m)
    lse = jnp.log(jnp.sum(e, axis=0, keepdims=True))
    o_ref[...] = jnp.where(valid, logits - m - lse, 0.0)


def _round_up(a, m):
    return ((a + m - 1) // m) * m


@jax.jit
def _forward(x, conv_w, conv_b, fc_w, fc_b):
    n = x.shape[0]
    xt = jnp.transpose(x.astype(jnp.float32), (1, 2, 3, 0)).reshape(IN_HW, n)

    nb = 512
    n_pad = _round_up(max(n, 2 * nb), nb)
    if n_pad != n:
        xt = jnp.pad(xt, ((0, 0), (0, n_pad - n)))

    cw = conv_w.astype(jnp.float32).reshape(C_OUT, N_TAPS)
    cb = conv_b.astype(jnp.float32)

    fcw_r = fc_w.astype(jnp.float32).reshape(N_CLASSES, C_OUT, OH, OW)
    fcw_p = jnp.pad(fcw_r, ((0, M_PAD - N_CLASSES), (0, 0), (0, 0),
                            (0, ROW_STRIDE - OW)))
    fcw_t = fcw_p.reshape(M_PAD, K_CAT)
    fcb_t = jnp.broadcast_to(
        jnp.pad(fc_b.astype(jnp.float32), (0, M_PAD - N_CLASSES))[:, None],
        (M_PAD, 128))

    grid = (n_pad // nb,)
    flops = n_pad * (2 * HW * N_TAPS * C_OUT + 2 * K_CAT * M_PAD)
    bytes_accessed = int(4 * (xt.size + cw.size + cb.size
                              + fcw_t.size + fcb_t.size + n_pad * M_PAD))

    out = pl.pallas_call(
        _cnn_kernel,
        out_shape=jax.ShapeDtypeStruct((M_PAD, n_pad), jnp.float32),
        grid=grid,
        in_specs=[
            pl.BlockSpec((IN_HW, nb), lambda i: (0, i)),
            pl.BlockSpec(memory_space=pltpu.MemorySpace.SMEM),
            pl.BlockSpec(memory_space=pltpu.MemorySpace.SMEM),
            pl.BlockSpec((M_PAD, K_CAT), lambda i: (0, 0)),
            pl.BlockSpec((M_PAD, 128), lambda i: (0, 0)),
        ],
        out_specs=pl.BlockSpec((M_PAD, nb), lambda i: (0, i)),
        scratch_shapes=[pltpu.VMEM((nb // 128, 30, 32, 128), jnp.float32),
                        pltpu.VMEM((K_CAT, nb), jnp.float32)],
        compiler_params=pltpu.CompilerParams(
            dimension_semantics=("parallel",)),
        cost_estimate=pl.CostEstimate(
            flops=flops,
            transcendentals=n_pad * M_PAD,
            bytes_accessed=bytes_accessed),
    )(xt, cw, cb, fcw_t, fcb_t)

    return out[:N_CLASSES, :n].T


def kernel(x, conv_w, conv_b, fc_w, fc_b):
    return _forward(x, conv_w, conv_b, fc_w, fc_b)


# speedup vs baseline: 9.9958x; 1.9124x over previous
import functools

import jax
import jax.numpy as jnp
from jax import lax
from jax.experimental import pallas as pl
from jax.experimental.pallas import tpu as pltpu

C_OUT = 10
OH = OW = 14
HW = OH * OW
IN_HW = 28 * 28
N_TAPS = 9
N_CLASSES = 10
ROW_STRIDE = 16
K_CAT = C_OUT * OH * ROW_STRIDE
M_PAD = 16


def _cnn_kernel(x_ref, cw_ref, cb_ref, fcw_ref, fcb_ref, o_ref,
                xp_ref, act_ref):
    nb = o_ref.shape[1]
    n_q = nb // 128

    xp_ref[...] = jnp.zeros(xp_ref.shape, jnp.float32)
    for q in range(n_q):
        for r in range(28):
            xp_ref[q, r + 1, 1:29, :] = x_ref[28 * r:28 * r + 28, q, :]

    zero2 = jnp.zeros((ROW_STRIDE - OW, 128), jnp.float32)
    for q in range(n_q):
        for i in range(OH):
            taps = [xp_ref[q, 2 * i + di, dj:dj + 27:2, :]
                    for di in range(3) for dj in range(3)]
            for c in range(C_OUT):
                acc = taps[0] * cw_ref[c, 0]
                for t in range(1, N_TAPS):
                    acc = acc + taps[t] * cw_ref[c, t]
                base = c * (OH * ROW_STRIDE) + i * ROW_STRIDE
                act_ref[pl.ds(base, OW), 128 * q:128 * (q + 1)] = (
                    jnp.maximum(acc + cb_ref[c], 0.0))
                act_ref[pl.ds(base + OW, ROW_STRIDE - OW),
                        128 * q:128 * (q + 1)] = zero2

    logits = jnp.dot(fcw_ref[...], act_ref[...],
                     preferred_element_type=jnp.float32) + fcb_ref[:, 0:1]

    row = lax.broadcasted_iota(jnp.int32, (M_PAD, nb), 0)
    valid = row < N_CLASSES
    masked = jnp.where(valid, logits, -jnp.inf)
    m = jnp.max(masked, axis=0, keepdims=True)
    e = jnp.exp(masked - m)
    lse = jnp.log(jnp.sum(e, axis=0, keepdims=True))
    o_ref[...] = jnp.where(valid, logits - m - lse, 0.0)


def _round_up(a, m):
    return ((a + m - 1) // m) * m


@jax.jit
def _forward(x, conv_w, conv_b, fc_w, fc_b):
    n = x.shape[0]
    xt = jnp.transpose(x.astype(jnp.float32), (1, 2, 3, 0)).reshape(IN_HW, n)

    nb = 1024
    n_pad = _round_up(max(n, 2 * nb), nb)
    if n_pad != n:
        xt = jnp.pad(xt, ((0, 0), (0, n_pad - n)))
    xt = xt.reshape(IN_HW, n_pad // 128, 128)

    cw = conv_w.astype(jnp.float32).reshape(C_OUT, N_TAPS)
    cb = conv_b.astype(jnp.float32)

    fcw_r = fc_w.astype(jnp.float32).reshape(N_CLASSES, C_OUT, OH, OW)
    fcw_p = jnp.pad(fcw_r, ((0, M_PAD - N_CLASSES), (0, 0), (0, 0),
                            (0, ROW_STRIDE - OW)))
    fcw_t = fcw_p.reshape(M_PAD, K_CAT)
    fcb_t = jnp.broadcast_to(
        jnp.pad(fc_b.astype(jnp.float32), (0, M_PAD - N_CLASSES))[:, None],
        (M_PAD, 128))

    grid = (n_pad // nb,)
    flops = n_pad * (2 * HW * N_TAPS * C_OUT + 2 * K_CAT * M_PAD)
    bytes_accessed = int(4 * (xt.size + cw.size + cb.size
                              + fcw_t.size + fcb_t.size + n_pad * M_PAD))

    out = pl.pallas_call(
        _cnn_kernel,
        out_shape=jax.ShapeDtypeStruct((M_PAD, n_pad), jnp.float32),
        grid=grid,
        in_specs=[
            pl.BlockSpec((IN_HW, nb // 128, 128), lambda i: (0, i, 0)),
            pl.BlockSpec(memory_space=pltpu.MemorySpace.SMEM),
            pl.BlockSpec(memory_space=pltpu.MemorySpace.SMEM),
            pl.BlockSpec((M_PAD, K_CAT), lambda i: (0, 0)),
            pl.BlockSpec((M_PAD, 128), lambda i: (0, 0)),
        ],
        out_specs=pl.BlockSpec((M_PAD, nb), lambda i: (0, i)),
        scratch_shapes=[pltpu.VMEM((nb // 128, 30, 32, 128), jnp.float32),
                        pltpu.VMEM((K_CAT, nb), jnp.float32)],
        compiler_params=pltpu.CompilerParams(
            dimension_semantics=("parallel",)),
        cost_estimate=pl.CostEstimate(
            flops=flops,
            transcendentals=n_pad * M_PAD,
            bytes_accessed=bytes_accessed),
    )(xt, cw, cb, fcw_t, fcb_t)

    return out[:N_CLASSES, :n].T


def kernel(x, conv_w, conv_b, fc_w, fc_b):
    return _forward(x, conv_w, conv_b, fc_w, fc_b)


# speedup vs baseline: 11.8014x; 1.1806x over previous
import functools

import jax
import jax.numpy as jnp
from jax import lax
from jax.experimental import pallas as pl
from jax.experimental.pallas import tpu as pltpu

C_OUT = 10
OH = OW = 14
HW = OH * OW
IN_HW = 28 * 28
N_TAPS = 9
N_CLASSES = 10
ROW_STRIDE = 16
CH_ROWS = OH * ROW_STRIDE
K_CAT = C_OUT * CH_ROWS
M_PAD = 16


def _cwt_builder_kernel(cw_ref, out_ref):
    c = pl.program_id(0)
    row = lax.broadcasted_iota(jnp.int32, (CH_ROWS, IN_HW), 0)
    col = lax.broadcasted_iota(jnp.int32, (CH_ROWS, IN_HW), 1)
    oi = row // ROW_STRIDE
    oj = row - ROW_STRIDE * oi
    pr = col // 28
    pc = col - 28 * pr
    valid = oj < OW
    acc = jnp.zeros((CH_ROWS, IN_HW), jnp.float32)
    for di in range(3):
        for dj in range(3):
            m = ((pr == 2 * oi + di - 1) & (pc == 2 * oj + dj - 1)) & valid
            acc = jnp.where(m, cw_ref[c, di * 3 + dj], acc)
    out_ref[...] = acc


def _cnn_kernel(x_ref, cwt_ref, cb_ref, fcw_ref, fcb_ref, o_ref,
                xf_ref, act_ref):
    nb = o_ref.shape[1]

    xf_ref[...] = x_ref[...].reshape(IN_HW, nb)

    for c in range(C_OUT):
        conv = jnp.dot(cwt_ref[c * CH_ROWS:(c + 1) * CH_ROWS, :],
                       xf_ref[...], preferred_element_type=jnp.float32)
        act_ref[c * CH_ROWS:(c + 1) * CH_ROWS, :] = (
            jnp.maximum(conv + cb_ref[c], 0.0))

    logits = jnp.dot(fcw_ref[...], act_ref[...],
                     preferred_element_type=jnp.float32) + fcb_ref[:, 0:1]

    row = lax.broadcasted_iota(jnp.int32, (M_PAD, nb), 0)
    valid = row < N_CLASSES
    masked = jnp.where(valid, logits, -jnp.inf)
    m = jnp.max(masked, axis=0, keepdims=True)
    e = jnp.exp(masked - m)
    lse = jnp.log(jnp.sum(e, axis=0, keepdims=True))
    o_ref[...] = jnp.where(valid, logits - m - lse, 0.0)


def _round_up(a, m):
    return ((a + m - 1) // m) * m


@jax.jit
def _forward(x, conv_w, conv_b, fc_w, fc_b):
    n = x.shape[0]
    xt = jnp.transpose(x.astype(jnp.float32), (1, 2, 3, 0)).reshape(IN_HW, n)

    nb = 1024
    n_pad = _round_up(max(n, 2 * nb), nb)
    if n_pad != n:
        xt = jnp.pad(xt, ((0, 0), (0, n_pad - n)))
    xt = xt.reshape(IN_HW, n_pad // 128, 128)

    cw = conv_w.astype(jnp.float32).reshape(C_OUT, N_TAPS)
    cb = conv_b.astype(jnp.float32)

    cwt = pl.pallas_call(
        _cwt_builder_kernel,
        out_shape=jax.ShapeDtypeStruct((K_CAT, IN_HW), jnp.float32),
        grid=(C_OUT,),
        in_specs=[pl.BlockSpec(memory_space=pltpu.MemorySpace.SMEM)],
        out_specs=pl.BlockSpec((CH_ROWS, IN_HW), lambda c: (c, 0)),
        compiler_params=pltpu.CompilerParams(
            dimension_semantics=("parallel",)),
    )(cw)

    fcw_r = fc_w.astype(jnp.float32).reshape(N_CLASSES, C_OUT, OH, OW)
    fcw_p = jnp.pad(fcw_r, ((0, M_PAD - N_CLASSES), (0, 0), (0, 0),
                            (0, ROW_STRIDE - OW)))
    fcw_t = fcw_p.reshape(M_PAD, K_CAT)
    fcb_t = jnp.broadcast_to(
        jnp.pad(fc_b.astype(jnp.float32), (0, M_PAD - N_CLASSES))[:, None],
        (M_PAD, 128))

    grid = (n_pad // nb,)
    flops = n_pad * (2 * IN_HW * K_CAT + 2 * K_CAT * M_PAD)
    bytes_accessed = int(4 * (xt.size + cwt.size + cb.size
                              + fcw_t.size + fcb_t.size + n_pad * M_PAD))

    out = pl.pallas_call(
        _cnn_kernel,
        out_shape=jax.ShapeDtypeStruct((M_PAD, n_pad), jnp.float32),
        grid=grid,
        in_specs=[
            pl.BlockSpec((IN_HW, nb // 128, 128), lambda i: (0, i, 0)),
            pl.BlockSpec((K_CAT, IN_HW), lambda i: (0, 0)),
            pl.BlockSpec(memory_space=pltpu.MemorySpace.SMEM),
            pl.BlockSpec((M_PAD, K_CAT), lambda i: (0, 0)),
            pl.BlockSpec((M_PAD, 128), lambda i: (0, 0)),
        ],
        out_specs=pl.BlockSpec((M_PAD, nb), lambda i: (0, i)),
        scratch_shapes=[pltpu.VMEM((IN_HW, nb), jnp.float32),
                        pltpu.VMEM((K_CAT, nb), jnp.float32)],
        compiler_params=pltpu.CompilerParams(
            dimension_semantics=("parallel",)),
        cost_estimate=pl.CostEstimate(
            flops=flops,
            transcendentals=n_pad * M_PAD,
            bytes_accessed=bytes_accessed),
    )(xt, cwt, cb, fcw_t, fcb_t)

    return out[:N_CLASSES, :n].T


def kernel(x, conv_w, conv_b, fc_w, fc_b):
    return _forward(x, conv_w, conv_b, fc_w, fc_b)


# speedup vs baseline: 33.1318x; 2.8074x over previous
import functools

import numpy as np

import jax
import jax.numpy as jnp
from jax import lax
from jax.experimental import pallas as pl
from jax.experimental.pallas import tpu as pltpu

C_OUT = 10
OH = OW = 14
HW = OH * OW
IN_HW = 28 * 28
N_TAPS = 9
N_CLASSES = 10
ROW_STRIDE = 16
I_ROWS = C_OUT * ROW_STRIDE
K_CAT = OH * I_ROWS
M_PAD = 16
W3_K = 3 * 28


def _w3_selection():
    s = np.zeros((N_TAPS, ROW_STRIDE, W3_K), np.float32)
    for dr in range(3):
        for dj in range(3):
            t = dr * 3 + dj
            for j in range(OW):
                cc = 2 * j + dj - 1
                if not 0 <= cc < 28:
                    continue
                s[t, j, dr * 28 + cc] = 1.0
    return s


_W3_SEL = _w3_selection()


def _cnn_kernel(x_ref, w3_ref, cb_ref, fcw_ref, fcb_ref, o_ref,
                xf_ref, act_ref):
    nb = o_ref.shape[1]

    xf_ref[...] = x_ref[...].reshape(IN_HW, nb)

    crow = lax.broadcasted_iota(jnp.int32, (I_ROWS, 128), 0) // ROW_STRIDE
    bias = jnp.zeros((I_ROWS, 128), jnp.float32)
    for c in range(C_OUT):
        bias = jnp.where(crow == c, cb_ref[c], bias)
    bias = bias[:, 0:1]

    for i in range(OH):
        if i == 0:
            conv = jnp.dot(w3_ref[:, 28:], xf_ref[0:56, :],
                           preferred_element_type=jnp.float32)
        else:
            conv = jnp.dot(w3_ref[...],
                           xf_ref[pl.ds(28 * (2 * i - 1), W3_K), :],
                           preferred_element_type=jnp.float32)
        act_ref[pl.ds(i * I_ROWS, I_ROWS), :] = (
            jnp.maximum(conv + bias, 0.0))

    logits = jnp.dot(fcw_ref[...], act_ref[...],
                     preferred_element_type=jnp.float32) + fcb_ref[:, 0:1]

    row = lax.broadcasted_iota(jnp.int32, (M_PAD, nb), 0)
    valid = row < N_CLASSES
    masked = jnp.where(valid, logits, -jnp.inf)
    m = jnp.max(masked, axis=0, keepdims=True)
    e = jnp.exp(masked - m)
    lse = jnp.log(jnp.sum(e, axis=0, keepdims=True))
    o_ref[...] = jnp.where(valid, logits - m - lse, 0.0)


def _round_up(a, m):
    return ((a + m - 1) // m) * m


@jax.jit
def _forward(x, conv_w, conv_b, fc_w, fc_b):
    n = x.shape[0]
    xt = jnp.transpose(x.astype(jnp.float32), (1, 2, 3, 0)).reshape(IN_HW, n)

    nb = 1024
    n_pad = _round_up(max(n, 2 * nb), nb)
    if n_pad != n:
        xt = jnp.pad(xt, ((0, 0), (0, n_pad - n)))
    xt = xt.reshape(IN_HW, n_pad // 128, 128)

    cw = conv_w.astype(jnp.float32).reshape(C_OUT, N_TAPS)
    cb = conv_b.astype(jnp.float32)
    w3 = jnp.einsum('ct,tjk->cjk', cw,
                    jnp.asarray(_W3_SEL)).reshape(I_ROWS, W3_K)

    fcw_r = jnp.transpose(
        fc_w.astype(jnp.float32).reshape(N_CLASSES, C_OUT, OH, OW),
        (0, 2, 1, 3))
    fcw_p = jnp.pad(fcw_r, ((0, M_PAD - N_CLASSES), (0, 0), (0, 0),
                            (0, ROW_STRIDE - OW)))
    fcw_t = fcw_p.reshape(M_PAD, K_CAT)
    fcb_t = jnp.broadcast_to(
        jnp.pad(fc_b.astype(jnp.float32), (0, M_PAD - N_CLASSES))[:, None],
        (M_PAD, 128))

    grid = (n_pad // nb,)
    flops = n_pad * (2 * HW * N_TAPS * C_OUT + 2 * K_CAT * M_PAD)
    bytes_accessed = int(4 * (xt.size + w3.size + cb.size
                              + fcw_t.size + fcb_t.size + n_pad * M_PAD))

    out = pl.pallas_call(
        _cnn_kernel,
        out_shape=jax.ShapeDtypeStruct((M_PAD, n_pad), jnp.float32),
        grid=grid,
        in_specs=[
            pl.BlockSpec((IN_HW, nb // 128, 128), lambda i: (0, i, 0)),
            pl.BlockSpec((I_ROWS, W3_K), lambda i: (0, 0)),
            pl.BlockSpec(memory_space=pltpu.MemorySpace.SMEM),
            pl.BlockSpec((M_PAD, K_CAT), lambda i: (0, 0)),
            pl.BlockSpec((M_PAD, 128), lambda i: (0, 0)),
        ],
        out_specs=pl.BlockSpec((M_PAD, nb), lambda i: (0, i)),
        scratch_shapes=[pltpu.VMEM((IN_HW, nb), jnp.float32),
                        pltpu.VMEM((K_CAT, nb), jnp.float32)],
        compiler_params=pltpu.CompilerParams(
            dimension_semantics=("parallel",)),
        cost_estimate=pl.CostEstimate(
            flops=flops,
            transcendentals=n_pad * M_PAD,
            bytes_accessed=bytes_accessed),
    )(xt, w3, cb, fcw_t, fcb_t)

    return out[:N_CLASSES, :n].T


def kernel(x, conv_w, conv_b, fc_w, fc_b):
    return _forward(x, conv_w, conv_b, fc_w, fc_b)
